# Initial kernel scaffold; baseline (speedup 1.0000x reference)
import jax
import jax.numpy as jnp
from jax.experimental import pallas as pl


def kernel(x, adj_t, edge_attr, emb_ea, W1, b1, W2, b2, W3, b3):
    raise NotImplementedError("write your pallas kernel here")


# speedup vs baseline: 3.3712x; 3.3712x over previous
import functools

import jax
import jax.numpy as jnp
from jax import lax
from jax.experimental import pallas as pl
from jax.experimental.pallas import tpu as pltpu
from jax.experimental.pallas import tpu_sc as plsc

N = 10000
E = 160000
D = 256
H = 128
DE = 16

NCORES = 2
NSUB = 16
NP = 10240
RT = NP // NSUB
EB = 128
NBLK = E // EB

_MESH = plsc.VectorSubcoreMesh(core_axis_name="c", subcore_axis_name="s")


def _blk_range(worker, nworkers):
    per = NBLK // nworkers
    rem = NBLK % nworkers
    start = worker * per + jnp.minimum(worker, rem)
    cnt = per + jnp.where(worker < rem, 1, 0)
    return start, cnt


def _newton_rsqrt(d):
    bits = lax.bitcast_convert_type(d, jnp.int32)
    y = lax.bitcast_convert_type(jnp.int32(0x5F3759DF) - (bits >> 1), jnp.float32)
    for _ in range(4):
        y = y * (1.5 - 0.5 * d * y * y)
    return jnp.where(d >= 0.5, y, 0.0)


def _pre_body(row2_h, col2_h, ea_h, dis_out, sp_out,
              zbuf, degbuf, dischunk, onesb, rowbuf, colbuf, normb,
              drbuf, eabuf, sbuf, sem, acc16):
    core = lax.axis_index("c")
    tile = lax.axis_index("s")
    r0 = tile * RT
    zero16 = jnp.zeros((16,), jnp.float32)
    one16 = jnp.ones((16,), jnp.float32)
    lanes = lax.broadcasted_iota(jnp.int32, (16,), 0)

    def _z(i, _):
        zbuf[i, :] = zero16
        onesb[i, :] = one16
        return 0
    lax.fori_loop(0, EB, _z, 0)

    def _z2(i, _):
        zbuf[EB + i, :] = zero16
        return 0
    lax.fori_loop(0, RT - EB, _z2, 0)
    pltpu.sync_copy(zbuf, acc16.at[pl.ds(r0, RT)])
    plsc.subcore_barrier()

    dstart, dcnt = _blk_range(tile, NSUB)

    def _deg(j, _):
        b = dstart + j
        pltpu.sync_copy(col2_h.at[b], colbuf)
        pltpu.sync_copy(onesb, acc16.at[colbuf], add=True)
        return 0
    lax.fori_loop(0, dcnt, _deg, 0)
    plsc.subcore_barrier()

    pltpu.sync_copy(acc16.at[pl.ds(r0, RT)], degbuf)

    def _nw(g, _):
        d = jnp.zeros((16,), jnp.float32)
        for j in range(16):
            v = degbuf[g * 16 + j, :]
            d = jnp.where(lanes == j, v, d)
        dischunk[pl.ds(g * 16, 16)] = _newton_rsqrt(d)
        return 0
    lax.fori_loop(0, RT // 16, _nw, 0)
    pltpu.sync_copy(zbuf, acc16.at[pl.ds(r0, RT)])
    pltpu.sync_copy(dischunk, dis_out.at[pl.ds(core * NP + r0, RT)])
    plsc.subcore_barrier()

    w = core * NSUB + tile
    sstart, scnt = _blk_range(w, NCORES * NSUB)
    doff = core * NP

    def _sblk(j, _):
        b = sstart + j
        pltpu.sync_copy(row2_h.at[b], rowbuf)
        pltpu.sync_copy(col2_h.at[b], colbuf)
        pltpu.sync_copy(ea_h.at[pl.ds(b * EB, EB)], eabuf)

        def _adj(g, _):
            rowbuf[pl.ds(g * 16, 16)] = rowbuf[pl.ds(g * 16, 16)] + doff
            return 0
        lax.fori_loop(0, EB // 16, _adj, 0)
        pltpu.async_copy(dis_out.at[rowbuf], drbuf, sem).wait()

        def _ng(g, _):
            normb[pl.ds(g * 16, 16)] = drbuf[pl.ds(g * 16, 16)]
            return 0
        lax.fori_loop(0, EB // 16, _ng, 0)

        def _pe(e, _):
            nv = normb[pl.ds(e, 16)][0]
            sbuf[e, :] = eabuf[e, :] * nv
            return 0
        lax.fori_loop(0, EB, _pe, 0)
        pltpu.sync_copy(sbuf, acc16.at[colbuf], add=True)
        return 0
    lax.fori_loop(0, scnt, _sblk, 0)
    plsc.subcore_barrier()

    pltpu.sync_copy(acc16.at[pl.ds(r0, RT)], degbuf)
    pltpu.sync_copy(degbuf, sp_out.at[pl.ds(core * NP + r0, RT)])


_pre_kernel = pl.kernel(
    _pre_body,
    out_type=(jax.ShapeDtypeStruct((2 * NP,), jnp.float32),
              jax.ShapeDtypeStruct((2 * NP, DE), jnp.float32)),
    mesh=_MESH,
    scratch_types=[
        pltpu.VMEM((RT, 16), jnp.float32),
        pltpu.VMEM((RT, 16), jnp.float32),
        pltpu.VMEM((RT,), jnp.float32),
        pltpu.VMEM((EB, 16), jnp.float32),
        pltpu.VMEM((EB,), jnp.int32),
        pltpu.VMEM((EB,), jnp.int32),
        pltpu.VMEM((EB + 16,), jnp.float32),
        pltpu.VMEM((EB,), jnp.float32),
        pltpu.VMEM((EB, DE), jnp.float32),
        pltpu.VMEM((EB, DE), jnp.float32),
        pltpu.SemaphoreType.DMA,
        pltpu.VMEM_SHARED((NP, 16), jnp.float32),
    ],
    compiler_params=pltpu.CompilerParams(use_tc_tiling_on_sc=False),
)


HQ = H // 2
NQ = 4


def _spmm_body(hs_h, row2_h, col2_h, out_h, rowbuf, colbuf, gbuf, zbuf, acc,
               sem):
    core = lax.axis_index("c")
    tile = lax.axis_index("s")
    r0 = tile * RT
    zero16 = jnp.zeros((16,), jnp.float32)

    def _z(i, _):
        for j in range(HQ // 16):
            zbuf[i, pl.ds(j * 16, 16)] = zero16
        return 0
    lax.fori_loop(0, EB, _z, 0)

    bstart, bcnt = _blk_range(tile, NSUB)
    for s in range(2):
        q = 2 * core + s
        roff = q * NP
        for half in range(RT // EB):
            pltpu.sync_copy(zbuf, acc.at[pl.ds(r0 + half * EB, EB)])
        plsc.subcore_barrier()

        def _blk(j, _):
            b = bstart + j
            pltpu.sync_copy(row2_h.at[b], rowbuf)
            pltpu.sync_copy(col2_h.at[b], colbuf)

            def _adj(g, _):
                v = rowbuf[pl.ds(g * 16, 16)]
                rowbuf[pl.ds(g * 16, 16)] = v + roff
                return 0
            lax.fori_loop(0, EB // 16, _adj, 0)
            pltpu.async_copy(hs_h.at[rowbuf], gbuf, sem).wait()
            pltpu.sync_copy(gbuf, acc.at[colbuf], add=True)
            return 0
        lax.fori_loop(0, bcnt, _blk, 0)
        plsc.subcore_barrier()

        for half in range(RT // EB):
            pltpu.sync_copy(acc.at[pl.ds(r0 + half * EB, EB)], gbuf)
            pltpu.sync_copy(gbuf, out_h.at[pl.ds(roff + r0 + half * EB, EB)])
        plsc.subcore_barrier()


_spmm_kernel = pl.kernel(
    _spmm_body,
    out_type=jax.ShapeDtypeStruct((NQ * NP, HQ), jnp.float32),
    mesh=_MESH,
    scratch_types=[
        pltpu.VMEM((EB,), jnp.int32),
        pltpu.VMEM((EB,), jnp.int32),
        pltpu.VMEM((EB, HQ), jnp.float32),
        pltpu.VMEM((EB, HQ), jnp.float32),
        pltpu.VMEM_SHARED((NP, HQ), jnp.float32),
        pltpu.SemaphoreType.DMA,
    ],
    compiler_params=pltpu.CompilerParams(use_tc_tiling_on_sc=False),
)


BN = 640


def _hq_specs(n):
    return [pl.BlockSpec((BN, HQ), lambda i: (i, 0)) for _ in range(n)]


def _hq_shapes(n):
    return [jax.ShapeDtypeStruct((NP, HQ), jnp.float32) for _ in range(n)]


def _prep_body(sp0, sp1, emb, dis, x, c_o, *xs_o):
    s = sp0[...] + sp1[...]
    c_o[...] = jnp.dot(s, emb[...], preferred_element_type=jnp.float32, precision=lax.Precision.HIGHEST) * dis[...]
    xs = x[...] * dis[...]
    for q in range(NQ):
        xs_o[q][...] = xs[:, q * HQ:(q + 1) * HQ]


def _make_prep():
    return pl.pallas_call(
        _prep_body,
        grid=(NP // BN,),
        in_specs=[
            pl.BlockSpec((BN, DE), lambda i: (i, 0)),
            pl.BlockSpec((BN, DE), lambda i: (i, 0)),
            pl.BlockSpec((DE, D), lambda i: (0, 0)),
            pl.BlockSpec((BN, 1), lambda i: (i, 0)),
            pl.BlockSpec((BN, D), lambda i: (i, 0)),
        ],
        out_specs=[pl.BlockSpec((BN, D), lambda i: (i, 0))] + _hq_specs(NQ),
        out_shape=[jax.ShapeDtypeStruct((NP, D), jnp.float32)] + _hq_shapes(NQ),
    )


def _layer_body(a0, a1, a2, a3, c, h, dis, W, b, hn_o, *hs_o):
    dd = dis[...]
    u = jnp.concatenate(
        [a0[...] * dd, a1[...] * dd, a2[...] * dd, a3[...] * dd], axis=1)
    u = u + c[...] + h[...]
    z = jnp.dot(u, W[...], preferred_element_type=jnp.float32, precision=lax.Precision.HIGHEST) + b[...]
    hn = jnp.maximum(z, 0.0)
    hn_o[...] = hn
    hh = hn * dd
    for q in range(NQ):
        hs_o[q][...] = hh[:, q * HQ:(q + 1) * HQ]


def _make_layer():
    return pl.pallas_call(
        _layer_body,
        grid=(NP // BN,),
        in_specs=_hq_specs(NQ) + [
            pl.BlockSpec((BN, D), lambda i: (i, 0)),
            pl.BlockSpec((BN, D), lambda i: (i, 0)),
            pl.BlockSpec((BN, 1), lambda i: (i, 0)),
            pl.BlockSpec((D, D), lambda i: (0, 0)),
            pl.BlockSpec((1, D), lambda i: (0, 0)),
        ],
        out_specs=[pl.BlockSpec((BN, D), lambda i: (i, 0))] + _hq_specs(NQ),
        out_shape=[jax.ShapeDtypeStruct((NP, D), jnp.float32)] + _hq_shapes(NQ),
    )


def _layer3_body(a0, a1, a2, a3, c, h, dis, W, b, hn_o):
    dd = dis[...]
    u = jnp.concatenate(
        [a0[...] * dd, a1[...] * dd, a2[...] * dd, a3[...] * dd], axis=1)
    u = u + c[...] + h[...]
    z = jnp.dot(u, W[...], preferred_element_type=jnp.float32, precision=lax.Precision.HIGHEST) + b[...]
    hn_o[...] = jnp.maximum(z, 0.0)


def _make_layer3():
    return pl.pallas_call(
        _layer3_body,
        grid=(NP // BN,),
        in_specs=_hq_specs(NQ) + [
            pl.BlockSpec((BN, D), lambda i: (i, 0)),
            pl.BlockSpec((BN, D), lambda i: (i, 0)),
            pl.BlockSpec((BN, 1), lambda i: (i, 0)),
            pl.BlockSpec((D, H), lambda i: (0, 0)),
            pl.BlockSpec((1, H), lambda i: (0, 0)),
        ],
        out_specs=pl.BlockSpec((BN, H), lambda i: (i, 0)),
        out_shape=jax.ShapeDtypeStruct((NP, H), jnp.float32),
    )


@jax.jit
def kernel(x, adj_t, edge_attr, emb_ea, W1, b1, W2, b2, W3, b3):
    row = adj_t[0]
    col = adj_t[1]
    row2 = row.reshape(NBLK, EB)
    col2 = col.reshape(NBLK, EB)

    dis, sp = _pre_kernel(row2, col2, edge_attr)
    dis2 = dis[:NP].reshape(NP, 1)
    xp = jnp.zeros((NP, D), jnp.float32).at[:N].set(x)

    c, *xs = _make_prep()(sp[:NP], sp[NP:], emb_ea, dis2, xp)

    h = xp
    hs = jnp.concatenate(xs, axis=0)
    layer_fn = _make_layer()
    for (W, b) in ((W1, b1), (W2, b2)):
        agg = _spmm_kernel(hs, row2, col2)
        aq = [agg[q * NP:(q + 1) * NP] for q in range(NQ)]
        h, *hq = layer_fn(*aq, c, h, dis2, W, b.reshape(1, D))
        hs = jnp.concatenate(hq, axis=0)
    agg = _spmm_kernel(hs, row2, col2)
    aq = [agg[q * NP:(q + 1) * NP] for q in range(NQ)]
    out = _make_layer3()(*aq, c, h, dis2, W3, b3.reshape(1, H))
    return out[:N]


# speedup vs baseline: 3.4477x; 1.0227x over previous
import functools

import jax
import jax.numpy as jnp
from jax import lax
from jax.experimental import pallas as pl
from jax.experimental.pallas import tpu as pltpu
from jax.experimental.pallas import tpu_sc as plsc

N = 10000
E = 160000
D = 256
H = 128
DE = 16

NCORES = 2
NSUB = 16
NP = 10240
RT = NP // NSUB
EB = 128
EP = 163840
NBLKP = EP // EB
BPT = NBLKP // NSUB
BPW = NBLKP // (NCORES * NSUB)

_MESH = plsc.VectorSubcoreMesh(core_axis_name="c", subcore_axis_name="s")


def _newton_rsqrt(d):
    bits = lax.bitcast_convert_type(d, jnp.int32)
    y = lax.bitcast_convert_type(jnp.int32(0x5F3759DF) - (bits >> 1), jnp.float32)
    for _ in range(4):
        y = y * (1.5 - 0.5 * d * y * y)
    return jnp.where(d >= 0.5, y, 0.0)


def _pre_body(row2_h, col2_h, ea_h, dis_out, sp_out,
              zbuf, degbuf, dischunk, onesb, rowbuf, colbuf, normb,
              drbuf, eabuf, sbuf, sem, acc16):
    core = lax.axis_index("c")
    tile = lax.axis_index("s")
    r0 = tile * RT
    zero16 = jnp.zeros((16,), jnp.float32)
    one16 = jnp.ones((16,), jnp.float32)
    lanes = lax.broadcasted_iota(jnp.int32, (16,), 0)

    def _z(i, _):
        zbuf[i, :] = zero16
        onesb[i, :] = one16
        return 0
    lax.fori_loop(0, EB, _z, 0)

    def _z2(i, _):
        zbuf[EB + i, :] = zero16
        return 0
    lax.fori_loop(0, RT - EB, _z2, 0)
    pltpu.sync_copy(zbuf, acc16.at[pl.ds(r0, RT)])
    plsc.subcore_barrier()

    dstart = tile * BPT

    def _deg(j, _):
        b = dstart + j
        pltpu.sync_copy(col2_h.at[b], colbuf)
        pltpu.sync_copy(onesb, acc16.at[colbuf], add=True)
        return 0
    lax.fori_loop(0, BPT, _deg, 0)
    plsc.subcore_barrier()

    pltpu.sync_copy(acc16.at[pl.ds(r0, RT)], degbuf)

    def _nw(g, _):
        d = jnp.zeros((16,), jnp.float32)
        for j in range(16):
            v = degbuf[g * 16 + j, :]
            d = jnp.where(lanes == j, v, d)
        dischunk[pl.ds(g * 16, 16)] = _newton_rsqrt(d)
        return 0
    lax.fori_loop(0, RT // 16, _nw, 0)
    pltpu.sync_copy(zbuf, acc16.at[pl.ds(r0, RT)])
    pltpu.sync_copy(dischunk, dis_out.at[pl.ds(core * NP + r0, RT)])
    plsc.subcore_barrier()

    w = core * NSUB + tile
    sstart = w * BPW
    doff = core * NP

    def _sblk(j, _):
        b = sstart + j
        pltpu.sync_copy(row2_h.at[b], rowbuf)
        pltpu.sync_copy(col2_h.at[b], colbuf)
        pltpu.sync_copy(ea_h.at[pl.ds(b * EB, EB)], eabuf)

        def _adj(g, _):
            rowbuf[pl.ds(g * 16, 16)] = rowbuf[pl.ds(g * 16, 16)] + doff
            return 0
        lax.fori_loop(0, EB // 16, _adj, 0)
        pltpu.async_copy(dis_out.at[rowbuf], drbuf, sem).wait()

        def _ng(g, _):
            normb[pl.ds(g * 16, 16)] = drbuf[pl.ds(g * 16, 16)]
            return 0
        lax.fori_loop(0, EB // 16, _ng, 0)

        def _pe(e, _):
            nv = normb[pl.ds(e, 16)][0]
            sbuf[e, :] = eabuf[e, :] * nv
            return 0
        lax.fori_loop(0, EB, _pe, 0)
        pltpu.sync_copy(sbuf, acc16.at[colbuf], add=True)
        return 0
    lax.fori_loop(0, BPW, _sblk, 0)
    plsc.subcore_barrier()

    pltpu.sync_copy(acc16.at[pl.ds(r0, RT)], degbuf)
    pltpu.sync_copy(degbuf, sp_out.at[pl.ds(core * NP + r0, RT)])


_pre_kernel = pl.kernel(
    _pre_body,
    out_type=(jax.ShapeDtypeStruct((2 * NP,), jnp.float32),
              jax.ShapeDtypeStruct((2 * NP, DE), jnp.float32)),
    mesh=_MESH,
    scratch_types=[
        pltpu.VMEM((RT, 16), jnp.float32),
        pltpu.VMEM((RT, 16), jnp.float32),
        pltpu.VMEM((RT,), jnp.float32),
        pltpu.VMEM((EB, 16), jnp.float32),
        pltpu.VMEM((EB,), jnp.int32),
        pltpu.VMEM((EB,), jnp.int32),
        pltpu.VMEM((EB + 16,), jnp.float32),
        pltpu.VMEM((EB,), jnp.float32),
        pltpu.VMEM((EB, DE), jnp.float32),
        pltpu.VMEM((EB, DE), jnp.float32),
        pltpu.SemaphoreType.DMA,
        pltpu.VMEM_SHARED((NP, 16), jnp.float32),
    ],
    compiler_params=pltpu.CompilerParams(use_tc_tiling_on_sc=False),
)


HQ = H // 2
NQ = 4


def _spmm_body(hs_h, row2_h, col2_h, out_h, rowbuf, colbuf, gbuf, zbuf, acc,
               sem_i, sem_g):
    core = lax.axis_index("c")
    tile = lax.axis_index("s")
    r0 = tile * RT
    zero16 = jnp.zeros((16,), jnp.float32)

    def _z(i, _):
        for j in range(HQ // 16):
            zbuf[i, pl.ds(j * 16, 16)] = zero16
        return 0
    lax.fori_loop(0, EB, _z, 0)

    bstart = tile * BPT

    def _fire_idx(b, s):
        pltpu.async_copy(row2_h.at[b], rowbuf.at[s], sem_i)
        pltpu.async_copy(col2_h.at[b], colbuf.at[s], sem_i)

    def _wait_idx(b, s):
        pltpu.make_async_copy(row2_h.at[b], rowbuf.at[s], sem_i).wait()
        pltpu.make_async_copy(col2_h.at[b], colbuf.at[s], sem_i).wait()

    for s in range(2):
        q = 2 * core + s
        roff = q * NP
        for half in range(RT // EB):
            pltpu.sync_copy(zbuf, acc.at[pl.ds(r0 + half * EB, EB)])
        plsc.subcore_barrier()

        _fire_idx(bstart, 0)
        _wait_idx(bstart, 0)

        def _adjust(slot, rof):
            def _adj(g, _):
                v = rowbuf[slot, pl.ds(g * 16, 16)]
                rowbuf[slot, pl.ds(g * 16, 16)] = v + rof
                return 0
            lax.fori_loop(0, EB // 16, _adj, 0)

        _adjust(0, roff)
        pltpu.async_copy(hs_h.at[rowbuf.at[0]], gbuf.at[0], sem_g)
        _fire_idx(bstart + 1, 1)

        def _pair(t, _):
            for slot in range(2):
                j = 2 * t + slot
                nb = bstart + j + 1

                @pl.when(j + 1 < BPT)
                def _():
                    _wait_idx(nb, 1 - slot)
                    _adjust(1 - slot, roff)
                pltpu.make_async_copy(
                    hs_h.at[rowbuf.at[slot]], gbuf.at[slot], sem_g).wait()

                @pl.when(j + 1 < BPT)
                def _():
                    pltpu.async_copy(
                        hs_h.at[rowbuf.at[1 - slot]], gbuf.at[1 - slot], sem_g)
                pltpu.sync_copy(gbuf.at[slot], acc.at[colbuf.at[slot]],
                                add=True)

                @pl.when(j + 2 < BPT)
                def _():
                    _fire_idx(nb + 1, slot)
            return 0
        lax.fori_loop(0, BPT // 2, _pair, 0)
        plsc.subcore_barrier()

        for half in range(RT // EB):
            pltpu.sync_copy(acc.at[pl.ds(r0 + half * EB, EB)], gbuf.at[0])
            pltpu.sync_copy(gbuf.at[0], out_h.at[pl.ds(roff + r0 + half * EB, EB)])
        plsc.subcore_barrier()


_spmm_kernel = pl.kernel(
    _spmm_body,
    out_type=jax.ShapeDtypeStruct((NQ * NP, HQ), jnp.float32),
    mesh=_MESH,
    scratch_types=[
        pltpu.VMEM((2, EB), jnp.int32),
        pltpu.VMEM((2, EB), jnp.int32),
        pltpu.VMEM((2, EB, HQ), jnp.float32),
        pltpu.VMEM((EB, HQ), jnp.float32),
        pltpu.VMEM_SHARED((NP, HQ), jnp.float32),
        pltpu.SemaphoreType.DMA,
        pltpu.SemaphoreType.DMA,
    ],
    compiler_params=pltpu.CompilerParams(use_tc_tiling_on_sc=False),
)


BN = 640


def _hq_specs(n):
    return [pl.BlockSpec((BN, HQ), lambda i: (i, 0)) for _ in range(n)]


def _hq_shapes(n):
    return [jax.ShapeDtypeStruct((NP, HQ), jnp.float32) for _ in range(n)]


def _prep_body(sp0, sp1, emb, dis, x, c_o, *xs_o):
    s = sp0[...] + sp1[...]
    c_o[...] = jnp.dot(s, emb[...], preferred_element_type=jnp.float32, precision=lax.Precision.HIGHEST) * dis[...]
    xs = x[...] * dis[...]
    for q in range(NQ):
        xs_o[q][...] = xs[:, q * HQ:(q + 1) * HQ]


def _make_prep():
    return pl.pallas_call(
        _prep_body,
        grid=(NP // BN,),
        in_specs=[
            pl.BlockSpec((BN, DE), lambda i: (i, 0)),
            pl.BlockSpec((BN, DE), lambda i: (i, 0)),
            pl.BlockSpec((DE, D), lambda i: (0, 0)),
            pl.BlockSpec((BN, 1), lambda i: (i, 0)),
            pl.BlockSpec((BN, D), lambda i: (i, 0)),
        ],
        out_specs=[pl.BlockSpec((BN, D), lambda i: (i, 0))] + _hq_specs(NQ),
        out_shape=[jax.ShapeDtypeStruct((NP, D), jnp.float32)] + _hq_shapes(NQ),
    )


def _layer_body(a0, a1, a2, a3, c, h, dis, W, b, hn_o, *hs_o):
    dd = dis[...]
    u = jnp.concatenate(
        [a0[...] * dd, a1[...] * dd, a2[...] * dd, a3[...] * dd], axis=1)
    u = u + c[...] + h[...]
    z = jnp.dot(u, W[...], preferred_element_type=jnp.float32, precision=lax.Precision.HIGHEST) + b[...]
    hn = jnp.maximum(z, 0.0)
    hn_o[...] = hn
    hh = hn * dd
    for q in range(NQ):
        hs_o[q][...] = hh[:, q * HQ:(q + 1) * HQ]


def _make_layer():
    return pl.pallas_call(
        _layer_body,
        grid=(NP // BN,),
        in_specs=_hq_specs(NQ) + [
            pl.BlockSpec((BN, D), lambda i: (i, 0)),
            pl.BlockSpec((BN, D), lambda i: (i, 0)),
            pl.BlockSpec((BN, 1), lambda i: (i, 0)),
            pl.BlockSpec((D, D), lambda i: (0, 0)),
            pl.BlockSpec((1, D), lambda i: (0, 0)),
        ],
        out_specs=[pl.BlockSpec((BN, D), lambda i: (i, 0))] + _hq_specs(NQ),
        out_shape=[jax.ShapeDtypeStruct((NP, D), jnp.float32)] + _hq_shapes(NQ),
    )


def _layer3_body(a0, a1, a2, a3, c, h, dis, W, b, hn_o):
    dd = dis[...]
    u = jnp.concatenate(
        [a0[...] * dd, a1[...] * dd, a2[...] * dd, a3[...] * dd], axis=1)
    u = u + c[...] + h[...]
    z = jnp.dot(u, W[...], preferred_element_type=jnp.float32, precision=lax.Precision.HIGHEST) + b[...]
    hn_o[...] = jnp.maximum(z, 0.0)


def _make_layer3():
    return pl.pallas_call(
        _layer3_body,
        grid=(NP // BN,),
        in_specs=_hq_specs(NQ) + [
            pl.BlockSpec((BN, D), lambda i: (i, 0)),
            pl.BlockSpec((BN, D), lambda i: (i, 0)),
            pl.BlockSpec((BN, 1), lambda i: (i, 0)),
            pl.BlockSpec((D, H), lambda i: (0, 0)),
            pl.BlockSpec((1, H), lambda i: (0, 0)),
        ],
        out_specs=pl.BlockSpec((BN, H), lambda i: (i, 0)),
        out_shape=jax.ShapeDtypeStruct((NP, H), jnp.float32),
    )


@jax.jit
def kernel(x, adj_t, edge_attr, emb_ea, W1, b1, W2, b2, W3, b3):
    pad = jnp.full((EP - E,), N, jnp.int32)
    row2 = jnp.concatenate([adj_t[0], pad]).reshape(NBLKP, EB)
    col2 = jnp.concatenate([adj_t[1], pad]).reshape(NBLKP, EB)
    eap = jnp.zeros((EP, DE), jnp.float32).at[:E].set(edge_attr)

    dis, sp = _pre_kernel(row2, col2, eap)
    dis2 = dis[:NP].reshape(NP, 1)
    xp = jnp.zeros((NP, D), jnp.float32).at[:N].set(x)

    c, *xs = _make_prep()(sp[:NP], sp[NP:], emb_ea, dis2, xp)

    h = xp
    hs = jnp.concatenate(xs, axis=0)
    layer_fn = _make_layer()
    for (W, b) in ((W1, b1), (W2, b2)):
        agg = _spmm_kernel(hs, row2, col2)
        aq = [agg[q * NP:(q + 1) * NP] for q in range(NQ)]
        h, *hq = layer_fn(*aq, c, h, dis2, W, b.reshape(1, D))
        hs = jnp.concatenate(hq, axis=0)
    agg = _spmm_kernel(hs, row2, col2)
    aq = [agg[q * NP:(q + 1) * NP] for q in range(NQ)]
    out = _make_layer3()(*aq, c, h, dis2, W3, b3.reshape(1, H))
    return out[:N]
